# baseline (speedup 1.0000x reference)
import dataclasses

import jax
import jax.numpy as jnp
from jax.experimental import pallas as pl
from jax.experimental.pallas import tpu as pltpu
from jax.experimental.pallas import tpu_sc as plsc

_CB = 128
_LANES = 16

_CP = pltpu.CompilerParams()
if "needs_layout_passes" in pltpu.CompilerParams.__dataclass_fields__:
    _CP = dataclasses.replace(_CP, needs_layout_passes=False)


def kernel(x, t_p):
    b, h, w, c = x.shape
    hw = h * w
    xr = jnp.reshape(x, (b, hw, c))
    tp1 = jnp.reshape(t_p, (hw * hw,))

    ncb = c // _CB
    ntask = b * ncb
    nworker = 32
    per_w = ntask // nworker
    ngrp = _CB // _LANES

    mesh = plsc.VectorSubcoreMesh(
        core_axis_name="core", subcore_axis_name="subcore")

    @pl.kernel(
        out_type=[
            jax.ShapeDtypeStruct((b, hw, c), jnp.float32),
            jax.ShapeDtypeStruct((b, hw, c), jnp.float32),
        ],
        mesh=mesh,
        scratch_types=[
            pltpu.VMEM((hw * hw,), jnp.float32),
            pltpu.VMEM((hw, _CB), jnp.float32),
            pltpu.VMEM((hw, _CB), jnp.float32),
            pltpu.VMEM((hw, _CB), jnp.float32),
            pltpu.SemaphoreType.DMA,
        ],
        compiler_params=_CP,
    )
    def sc_kernel(x_hbm, tp_hbm, m_hbm, t_hbm, table_v, x_v, t_v, m_v, sem):
        cid = jax.lax.axis_index("core")
        sid = jax.lax.axis_index("subcore")
        wid = cid * 16 + sid
        pltpu.async_copy(tp_hbm, table_v, sem).wait()

        @pl.loop(0, per_w)
        def _task(t):
            task = wid * per_w + t
            bi = task // ncb
            cb = (task % ncb) * _CB
            pltpu.async_copy(
                x_hbm.at[bi, :, pl.ds(cb, _CB)], x_v, sem).wait()

            def step_a(q, carry):
                mxs, ixs = carry
                new_mx, new_ix = [], []
                for g in range(ngrp):
                    v = x_v[q, pl.ds(g * _LANES, _LANES)]
                    gt = v > mxs[g]
                    qv = jnp.full((_LANES,), q, jnp.int32)
                    new_mx.append(jnp.where(gt, v, mxs[g]))
                    new_ix.append(jnp.where(gt, qv, ixs[g]))
                return (tuple(new_mx), tuple(new_ix))

            mx0 = tuple(x_v[0, pl.ds(g * _LANES, _LANES)] for g in range(ngrp))
            ix0 = tuple(jnp.zeros((_LANES,), jnp.int32) for _ in range(ngrp))
            mxs, ixs = jax.lax.fori_loop(1, hw, step_a, (mx0, ix0))

            bases = []
            for g in range(ngrp):
                ix = jnp.where(mxs[g] == 0.0, hw - 1, ixs[g])
                bases.append(ix * hw)

            def step_b(q, carry):
                for g in range(ngrp):
                    sl = pl.ds(g * _LANES, _LANES)
                    tv = plsc.load_gather(table_v, [bases[g] + q])
                    xv = x_v[q, sl]
                    t_v[q, sl] = tv
                    m_v[q, sl] = jnp.maximum(xv * tv, 0.0)
                return carry

            jax.lax.fori_loop(0, hw, step_b, 0)

            pltpu.async_copy(
                t_v, t_hbm.at[bi, :, pl.ds(cb, _CB)], sem).wait()
            pltpu.async_copy(
                m_v, m_hbm.at[bi, :, pl.ds(cb, _CB)], sem).wait()

    masked_r, tmpl_r = sc_kernel(xr, tp1)
    masked = jnp.reshape(masked_r, (b, h, w, c))
    templates = jnp.reshape(tmpl_r, (b, h, w, c))
    return (masked, x, templates)


# speedup vs baseline: 1.4105x; 1.4105x over previous
import dataclasses

import jax
import jax.numpy as jnp
from jax.experimental import pallas as pl
from jax.experimental.pallas import tpu as pltpu
from jax.experimental.pallas import tpu_sc as plsc

_SPLIT = 96
_NB = 8
_CB = 128
_LANES = 16

_CP = pltpu.CompilerParams()
if "needs_layout_passes" in pltpu.CompilerParams.__dataclass_fields__:
    _CP = dataclasses.replace(_CP, needs_layout_passes=False)


def _tc_kernel_body(x_ref, tpT_ref, masked_ref, tmpl_ref):
    tpT = tpT_ref[...]
    for n in range(_NB):
        xb = x_ref[n]
        hw, c = xb.shape
        mx = jnp.max(xb, axis=0)
        iota = jax.lax.broadcasted_iota(jnp.int32, (hw, c), 0)
        idx = jnp.min(jnp.where(xb == mx[None, :], iota, hw), axis=0)
        idx = jnp.where(mx == 0.0, hw - 1, idx)
        onehot = (iota == idx[None, :]).astype(jnp.float32)
        tmpl = jax.lax.dot_general(
            tpT, onehot, (((1,), (0,)), ((), ())),
            preferred_element_type=jnp.float32)
        tmpl_ref[n] = tmpl
        masked_ref[n] = jnp.maximum(xb * tmpl, 0.0)


def _tc_part(xr, tpT, nb_batches):
    hw, c = xr.shape[1], xr.shape[2]
    return pl.pallas_call(
        _tc_kernel_body,
        grid=(nb_batches // _NB,),
        in_specs=[
            pl.BlockSpec((_NB, hw, c), lambda i: (i, 0, 0)),
            pl.BlockSpec((hw, hw), lambda i: (0, 0)),
        ],
        out_specs=[
            pl.BlockSpec((_NB, hw, c), lambda i: (i, 0, 0)),
            pl.BlockSpec((_NB, hw, c), lambda i: (i, 0, 0)),
        ],
        out_shape=[
            jax.ShapeDtypeStruct((nb_batches, hw, c), jnp.float32),
            jax.ShapeDtypeStruct((nb_batches, hw, c), jnp.float32),
        ],
        compiler_params=pltpu.CompilerParams(
            dimension_semantics=("arbitrary",),
        ),
    )(xr, tpT)


def _sc_part(xr, tp1, start, nb_batches):
    b, hw, c = xr.shape
    ncb = c // _CB
    ntask = nb_batches * ncb
    nworker = 32
    per_w = ntask // nworker
    ngrp = _CB // _LANES

    mesh = plsc.VectorSubcoreMesh(
        core_axis_name="core", subcore_axis_name="subcore")

    @pl.kernel(
        out_type=[
            jax.ShapeDtypeStruct((nb_batches, hw, c), jnp.float32),
            jax.ShapeDtypeStruct((nb_batches, hw, c), jnp.float32),
        ],
        mesh=mesh,
        scratch_types=[
            pltpu.VMEM((hw * hw,), jnp.float32),
            pltpu.VMEM((hw, _CB), jnp.float32),
            pltpu.VMEM((hw, _CB), jnp.float32),
            pltpu.VMEM((hw, _CB), jnp.float32),
            pltpu.SemaphoreType.DMA,
        ],
        compiler_params=_CP,
    )
    def sc_kernel(x_hbm, tp_hbm, m_hbm, t_hbm, table_v, x_v, t_v, m_v, sem):
        cid = jax.lax.axis_index("core")
        sid = jax.lax.axis_index("subcore")
        wid = cid * 16 + sid
        pltpu.async_copy(tp_hbm, table_v, sem).wait()

        @pl.loop(0, per_w)
        def _task(t):
            task = wid * per_w + t
            bo = task // ncb
            bi = start + bo
            cb = (task % ncb) * _CB
            pltpu.async_copy(
                x_hbm.at[bi, :, pl.ds(cb, _CB)], x_v, sem).wait()

            def step_a(q, carry):
                mxs, ixs = carry
                new_mx, new_ix = [], []
                for g in range(ngrp):
                    v = x_v[q, pl.ds(g * _LANES, _LANES)]
                    gt = v > mxs[g]
                    qv = jnp.full((_LANES,), q, jnp.int32)
                    new_mx.append(jnp.where(gt, v, mxs[g]))
                    new_ix.append(jnp.where(gt, qv, ixs[g]))
                return (tuple(new_mx), tuple(new_ix))

            mx0 = tuple(x_v[0, pl.ds(g * _LANES, _LANES)]
                        for g in range(ngrp))
            ix0 = tuple(jnp.zeros((_LANES,), jnp.int32) for _ in range(ngrp))
            mxs, ixs = jax.lax.fori_loop(1, hw, step_a, (mx0, ix0))

            bases = []
            for g in range(ngrp):
                ix = jnp.where(mxs[g] == 0.0, hw - 1, ixs[g])
                bases.append(ix * hw)

            def step_b(q, carry):
                for g in range(ngrp):
                    sl = pl.ds(g * _LANES, _LANES)
                    tv = plsc.load_gather(table_v, [bases[g] + q])
                    xv = x_v[q, sl]
                    t_v[q, sl] = tv
                    m_v[q, sl] = jnp.maximum(xv * tv, 0.0)
                return carry

            jax.lax.fori_loop(0, hw, step_b, 0)

            pltpu.async_copy(
                t_v, t_hbm.at[bo, :, pl.ds(cb, _CB)], sem).wait()
            pltpu.async_copy(
                m_v, m_hbm.at[bo, :, pl.ds(cb, _CB)], sem).wait()

    return sc_kernel(xr, tp1)


def kernel(x, t_p):
    b, h, w, c = x.shape
    hw = h * w
    xr = jnp.reshape(x, (b, hw, c))
    tp_flat = jnp.reshape(t_p, (hw, hw))
    tpT = jnp.transpose(tp_flat, (1, 0))
    tp1 = jnp.reshape(t_p, (hw * hw,))

    m_tc, t_tc = _tc_part(xr, tpT, _SPLIT)
    m_sc, t_sc = _sc_part(xr, tp1, _SPLIT, b - _SPLIT)

    masked_r = jnp.concatenate([m_tc, m_sc], axis=0)
    tmpl_r = jnp.concatenate([t_tc, t_sc], axis=0)
    masked = jnp.reshape(masked_r, (b, h, w, c))
    templates = jnp.reshape(tmpl_r, (b, h, w, c))
    return (masked, x, templates)


# speedup vs baseline: 1.4112x; 1.0005x over previous
import dataclasses

import jax
import jax.numpy as jnp
from jax.experimental import pallas as pl
from jax.experimental.pallas import tpu as pltpu
from jax.experimental.pallas import tpu_sc as plsc

_SPLIT = 96
_NB = 8
_CB = 128
_LANES = 16

_CP = pltpu.CompilerParams()
if "needs_layout_passes" in pltpu.CompilerParams.__dataclass_fields__:
    _CP = dataclasses.replace(_CP, needs_layout_passes=False)


def _tc_kernel_body(x_ref, tpT_ref, masked_ref, tmpl_ref):
    tpT = tpT_ref[...]
    for n in range(_NB):
        xb = x_ref[n]
        hw, c = xb.shape
        mx = jnp.max(xb, axis=0)
        iota = jax.lax.broadcasted_iota(jnp.int32, (hw, c), 0)
        idx = jnp.min(jnp.where(xb == mx[None, :], iota, hw), axis=0)
        idx = jnp.where(mx == 0.0, hw - 1, idx)
        onehot = (iota == idx[None, :]).astype(jnp.float32)
        tmpl = jax.lax.dot_general(
            tpT, onehot, (((1,), (0,)), ((), ())),
            preferred_element_type=jnp.float32)
        tmpl_ref[n] = tmpl
        masked_ref[n] = jnp.maximum(xb * tmpl, 0.0)


def _tc_part(xr, tpT, nb_batches):
    hw, c = xr.shape[1], xr.shape[2]
    return pl.pallas_call(
        _tc_kernel_body,
        grid=(nb_batches // _NB,),
        in_specs=[
            pl.BlockSpec((_NB, hw, c), lambda i: (i, 0, 0)),
            pl.BlockSpec((hw, hw), lambda i: (0, 0)),
        ],
        out_specs=[
            pl.BlockSpec((_NB, hw, c), lambda i: (i, 0, 0)),
            pl.BlockSpec((_NB, hw, c), lambda i: (i, 0, 0)),
        ],
        out_shape=[
            jax.ShapeDtypeStruct((nb_batches, hw, c), jnp.float32),
            jax.ShapeDtypeStruct((nb_batches, hw, c), jnp.float32),
        ],
        compiler_params=pltpu.CompilerParams(
            dimension_semantics=("arbitrary",),
        ),
    )(xr, tpT)


def _sc_part(xr, tp1, start, nb_batches):
    b, hw, c = xr.shape
    ncb = c // _CB
    ntask = nb_batches * ncb
    nworker = 32
    per_w = ntask // nworker
    ngrp = _CB // _LANES

    mesh = plsc.VectorSubcoreMesh(
        core_axis_name="core", subcore_axis_name="subcore")

    @pl.kernel(
        out_type=[
            jax.ShapeDtypeStruct((nb_batches, hw, c), jnp.float32),
            jax.ShapeDtypeStruct((nb_batches, hw, c), jnp.float32),
        ],
        mesh=mesh,
        scratch_types=[
            pltpu.VMEM((hw * hw,), jnp.float32),
            pltpu.VMEM((hw, _CB), jnp.float32),
            pltpu.VMEM((hw, _CB), jnp.float32),
            pltpu.VMEM((hw, _CB), jnp.float32),
            pltpu.SemaphoreType.DMA,
        ],
        compiler_params=_CP,
    )
    def sc_kernel(x_hbm, tp_hbm, m_hbm, t_hbm, table_v, x_v, t_v, m_v, sem):
        cid = jax.lax.axis_index("core")
        sid = jax.lax.axis_index("subcore")
        wid = cid * 16 + sid
        pltpu.async_copy(tp_hbm, table_v, sem).wait()

        @pl.loop(0, per_w)
        def _task(t):
            task = wid * per_w + t
            bo = task // ncb
            bi = start + bo
            cb = (task % ncb) * _CB
            pltpu.async_copy(
                x_hbm.at[bi, :, pl.ds(cb, _CB)], x_v, sem).wait()

            def step_a(q, carry):
                mxs, ixs = carry
                new_mx, new_ix = [], []
                for g in range(ngrp):
                    v = x_v[q, pl.ds(g * _LANES, _LANES)]
                    gt = v > mxs[g]
                    qv = jnp.full((_LANES,), q, jnp.int32)
                    new_mx.append(jnp.where(gt, v, mxs[g]))
                    new_ix.append(jnp.where(gt, qv, ixs[g]))
                return (tuple(new_mx), tuple(new_ix))

            mx0 = tuple(x_v[0, pl.ds(g * _LANES, _LANES)]
                        for g in range(ngrp))
            ix0 = tuple(jnp.zeros((_LANES,), jnp.int32) for _ in range(ngrp))
            mxs, ixs = jax.lax.fori_loop(1, hw, step_a, (mx0, ix0))

            bases = []
            for g in range(ngrp):
                ix = jnp.where(mxs[g] == 0.0, hw - 1, ixs[g])
                bases.append(ix * hw)

            def step_b(q, carry):
                for g in range(ngrp):
                    sl = pl.ds(g * _LANES, _LANES)
                    tv = plsc.load_gather(table_v, [bases[g] + q])
                    xv = x_v[q, sl]
                    t_v[q, sl] = tv
                    m_v[q, sl] = jnp.maximum(xv * tv, 0.0)
                return carry

            jax.lax.fori_loop(0, hw, step_b, 0)

            pltpu.async_copy(
                t_v, t_hbm.at[bo, :, pl.ds(cb, _CB)], sem).wait()
            pltpu.async_copy(
                m_v, m_hbm.at[bo, :, pl.ds(cb, _CB)], sem).wait()

    return sc_kernel(xr, tp1)


def kernel(x, t_p):
    b, h, w, c = x.shape
    hw = h * w
    xr = jnp.reshape(x, (b, hw, c))
    tp_flat = jnp.reshape(t_p, (hw, hw))
    tpT = jnp.transpose(tp_flat, (1, 0))
    tp1 = jnp.reshape(t_p, (hw * hw,))

    m_sc, t_sc = _sc_part(xr, tp1, _SPLIT, b - _SPLIT)
    m_tc, t_tc = _tc_part(xr, tpT, _SPLIT)

    masked_r = jnp.concatenate([m_tc, m_sc], axis=0)
    tmpl_r = jnp.concatenate([t_tc, t_sc], axis=0)
    masked = jnp.reshape(masked_r, (b, h, w, c))
    templates = jnp.reshape(tmpl_r, (b, h, w, c))
    return (masked, x, templates)


# speedup vs baseline: 2.1454x; 1.5203x over previous
import jax
import jax.numpy as jnp
from jax.experimental import pallas as pl
from jax.experimental.pallas import tpu as pltpu

_NB = 8


def _masked_kernel(x_ref, tpT_ref, masked_ref, tmpl_ref):
    tpT = tpT_ref[...]
    for n in range(_NB):
        xb = x_ref[n]
        hw, c = xb.shape
        mx = jnp.max(xb, axis=0)
        iota = jax.lax.broadcasted_iota(jnp.int32, (hw, c), 0)
        idx = jnp.min(jnp.where(xb == mx[None, :], iota, hw), axis=0)
        idx = jnp.where(mx == 0.0, hw - 1, idx)
        onehot = (iota == idx[None, :]).astype(jnp.float32)
        tmpl = jax.lax.dot_general(
            tpT, onehot, (((1,), (0,)), ((), ())),
            preferred_element_type=jnp.float32)
        tmpl_ref[n] = tmpl
        masked_ref[n] = jnp.maximum(xb * tmpl, 0.0)


def kernel(x, t_p):
    b, h, w, c = x.shape
    hw = h * w
    xr = jnp.reshape(x, (b, hw, c))
    tpT = jnp.transpose(jnp.reshape(t_p, (hw, hw)), (1, 0))
    masked_r, tmpl_r = pl.pallas_call(
        _masked_kernel,
        grid=(b // _NB,),
        in_specs=[
            pl.BlockSpec((_NB, hw, c), lambda i: (i, 0, 0)),
            pl.BlockSpec((hw, hw), lambda i: (0, 0)),
        ],
        out_specs=[
            pl.BlockSpec((_NB, hw, c), lambda i: (i, 0, 0)),
            pl.BlockSpec((_NB, hw, c), lambda i: (i, 0, 0)),
        ],
        out_shape=[
            jax.ShapeDtypeStruct((b, hw, c), jnp.float32),
            jax.ShapeDtypeStruct((b, hw, c), jnp.float32),
        ],
        compiler_params=pltpu.CompilerParams(
            dimension_semantics=("arbitrary",),
        ),
    )(xr, tpT)
    masked = jnp.reshape(masked_r, (b, h, w, c))
    templates = jnp.reshape(tmpl_r, (b, h, w, c))
    return (masked, x, templates)
